# Initial kernel scaffold; baseline (speedup 1.0000x reference)
import jax
import jax.numpy as jnp
from jax.experimental import pallas as pl


def kernel(hidden_states, W_router, w1, w2):
    raise NotImplementedError("write your pallas kernel here")


# speedup vs baseline: 1.4774x; 1.4774x over previous
import functools

import jax
import jax.numpy as jnp
from jax import lax
from jax.experimental import pallas as pl
from jax.experimental.pallas import tpu as pltpu
from jax.experimental.pallas import tpu_sc as plsc

E = 8
H = 1024
FFN = 2048
F2 = 2 * FFN
T = 2048
K = 2
NPAIR = K * T

TM = 128
NT = 40
P = NT * TM

NC = 2
NS = 16
NW = NC * NS
HALF = P // NC
SLOTS_W = HALF // NS
GCH = 80
CPW = NPAIR // NW
CCH = 64


def _plan_body(x_ref, wr_ref, inv_ref, w01_ref, te_ref, tv_ref):
    x = x_ref[...]
    wr = wr_ref[...]
    mix = jnp.dot(x, wr, preferred_element_type=jnp.float32)
    q, k, v = mix[:, 0:E], mix[:, E:2 * E], mix[:, 2 * E:3 * E]
    cols = []
    for i in range(E):
        a = q[:, i:i + 1] * k
        m = jnp.max(a, axis=1, keepdims=True)
        ex = jnp.exp(a - m)
        cols.append(jnp.sum(ex * v, axis=1, keepdims=True)
                    / jnp.sum(ex, axis=1, keepdims=True))
    logits = jnp.concatenate(cols, axis=1)
    iota8 = lax.broadcasted_iota(jnp.int32, (T, E), 1)
    l0 = jnp.max(logits, axis=1, keepdims=True)
    i0 = jnp.min(jnp.where(logits == l0, iota8, E), axis=1, keepdims=True)
    rest = jnp.where(iota8 == i0, -jnp.inf, logits)
    l1 = jnp.max(rest, axis=1, keepdims=True)
    i1 = jnp.min(jnp.where(rest == l1, iota8, E), axis=1, keepdims=True)
    w0 = 1.0 / (1.0 + jnp.exp(l1 - l0))

    oh0 = (iota8 == i0).astype(jnp.float32)
    oh1 = (iota8 == i1).astype(jnp.float32)
    assign = oh0 + oh1
    r = lax.broadcasted_iota(jnp.int32, (TM, TM), 0)
    c = lax.broadcasted_iota(jnp.int32, (TM, TM), 1)
    tri = (r >= c).astype(jnp.float32)
    carry = jnp.zeros((1, E), jnp.float32)
    parts = []
    for b in range(T // TM):
        cum = jnp.dot(tri, assign[b * TM:(b + 1) * TM, :],
                      preferred_element_type=jnp.float32) + carry
        parts.append(cum)
        carry = cum[TM - 1:TM, :]
    incl = jnp.concatenate(parts, axis=0)
    cnt = carry
    tiles = jnp.ceil(cnt / TM)
    ue = (lax.broadcasted_iota(jnp.int32, (E, E), 0)
          <= lax.broadcasted_iota(jnp.int32, (E, E), 1)).astype(jnp.float32)
    cumt = jnp.dot(tiles, ue, preferred_element_type=jnp.float32)
    start_slot = (cumt - tiles) * TM
    pos = start_slot + incl - 1.0
    inv0 = jnp.sum(oh0 * pos, axis=1, keepdims=True)
    inv1 = jnp.sum(oh1 * pos, axis=1, keepdims=True)
    inv_ref[...] = jnp.concatenate([inv0, inv1], axis=1).astype(jnp.int32)
    w01_ref[...] = jnp.concatenate([w0, 1.0 - w0], axis=1)
    jt = lax.broadcasted_iota(jnp.int32, (NT, E), 0).astype(jnp.float32)
    raw = jnp.sum((cumt <= jt).astype(jnp.float32), axis=1, keepdims=True)
    last_active = jnp.sum((cumt < cumt[0:1, E - 1:E]).astype(jnp.float32),
                          axis=1, keepdims=True)
    te_ref[...] = jnp.minimum(raw, last_active).astype(jnp.int32)
    tv_ref[...] = (jt[:, 0:1] < cumt[0:1, E - 1:E]).astype(jnp.int32)


_plan = pl.pallas_call(
    _plan_body,
    out_shape=[
        jax.ShapeDtypeStruct((T, 2), jnp.int32),
        jax.ShapeDtypeStruct((T, 2), jnp.float32),
        jax.ShapeDtypeStruct((NT, 1), jnp.int32),
        jax.ShapeDtypeStruct((NT, 1), jnp.int32),
    ],
)


@functools.cache
def _sc_mesh():
    return plsc.VectorSubcoreMesh(
        core_axis_name="c", subcore_axis_name="s",
        num_cores=NC, num_subcores=NS)


@functools.cache
def _dispatch_kernel():
    @functools.partial(
        pl.kernel,
        out_type=jax.ShapeDtypeStruct((P, H), jnp.float32),
        mesh=_sc_mesh(),
        scratch_types=[
            pltpu.VMEM((NPAIR,), jnp.int32),
            pltpu.VMEM((NPAIR,), jnp.int32),
            pltpu.VMEM((HALF,), jnp.int32),
            pltpu.VMEM_SHARED((HALF,), jnp.int32),
            pltpu.VMEM((SLOTS_W,), jnp.int32),
            pltpu.VMEM((GCH, H), jnp.float32),
            pltpu.SemaphoreType.DMA,
        ],
        compiler_params=pltpu.CompilerParams(needs_layout_passes=False),
    )
    def dispatch(x_hbm, slots_hbm, toks_hbm, xs_hbm,
                 ps_v, pt_v, map_v, shared_v, idx_v, rows_v, sem):
        c = lax.axis_index("c")
        s = lax.axis_index("s")
        lo = c * HALF

        @pl.when(s == 0)
        def _():
            pltpu.sync_copy(slots_hbm, ps_v)
            pltpu.sync_copy(toks_hbm, pt_v)

            def zero_body(i, carry):
                map_v[pl.ds(i * 16, 16)] = jnp.zeros((16,), jnp.int32)
                return carry

            lax.fori_loop(0, HALF // 16, zero_body, 0)

            def scat_body(i, carry):
                idx = ps_v[pl.ds(i * 16, 16)]
                tok = pt_v[pl.ds(i * 16, 16)]
                rel = idx - lo
                m = (rel >= 0) & (rel < HALF)
                plsc.store_scatter(map_v, [rel], tok, mask=m)
                return carry

            lax.fori_loop(0, NPAIR // 16, scat_body, 0)
            pltpu.sync_copy(map_v, shared_v)

        plsc.subcore_barrier()
        base = s * SLOTS_W
        pltpu.sync_copy(shared_v.at[pl.ds(base, SLOTS_W)], idx_v)
        for ch in range(SLOTS_W // GCH):
            pltpu.async_copy(
                x_hbm.at[idx_v.at[pl.ds(ch * GCH, GCH)]], rows_v, sem).wait()
            pltpu.sync_copy(
                rows_v, xs_hbm.at[pl.ds(lo + base + ch * GCH, GCH)])

    return dispatch


def _ffn_body(te_ref, tv_ref, xs_ref, w1_ref, w2_ref, ys_ref):
    j = pl.program_id(0)

    @pl.when(tv_ref[j] == 1)
    def _():
        xb = xs_ref[...]
        h = jnp.dot(xb, w1_ref[0], preferred_element_type=jnp.float32)
        a = h[:, :FFN]
        b = h[:, FFN:]
        act = (a * lax.logistic(a)) * b
        ys_ref[...] = jnp.dot(act, w2_ref[0],
                              preferred_element_type=jnp.float32)


_ffn = pl.pallas_call(
    _ffn_body,
    grid_spec=pltpu.PrefetchScalarGridSpec(
        num_scalar_prefetch=2,
        grid=(NT,),
        in_specs=[
            pl.BlockSpec((TM, H), lambda j, te, tv: (j, 0)),
            pl.BlockSpec((1, H, F2), lambda j, te, tv: (te[j], 0, 0)),
            pl.BlockSpec((1, FFN, H), lambda j, te, tv: (te[j], 0, 0)),
        ],
        out_specs=pl.BlockSpec((TM, H), lambda j, te, tv: (j, 0)),
    ),
    out_shape=jax.ShapeDtypeStruct((P, H), jnp.float32),
    compiler_params=pltpu.CompilerParams(
        dimension_semantics=("arbitrary",)),
)


@functools.cache
def _combine_kernel():
    @functools.partial(
        pl.kernel,
        out_type=jax.ShapeDtypeStruct((NPAIR, H), jnp.float32),
        mesh=_sc_mesh(),
        scratch_types=[
            pltpu.VMEM((CPW,), jnp.int32),
            pltpu.VMEM((CCH, H), jnp.float32),
            pltpu.SemaphoreType.DMA,
        ],
    )
    def combine(ys_hbm, slots_hbm, g_hbm, idx_v, rows_v, sem):
        c = lax.axis_index("c")
        s = lax.axis_index("s")
        base = (s * NC + c) * CPW
        pltpu.sync_copy(slots_hbm.at[pl.ds(base, CPW)], idx_v)
        for ch in range(CPW // CCH):
            pltpu.async_copy(
                ys_hbm.at[idx_v.at[pl.ds(ch * CCH, CCH)]],
                rows_v, sem).wait()
            pltpu.sync_copy(rows_v, g_hbm.at[pl.ds(base + ch * CCH, CCH)])

    return combine


def _mix_body(g_ref, gg_ref, w_ref, o_ref):
    w = w_ref[...]
    o_ref[...] = g_ref[...] * w[:, 0:1] + gg_ref[...] * w[:, 1:2]


_MIX_TB = 256
_mix = pl.pallas_call(
    _mix_body,
    grid=(T // _MIX_TB,),
    in_specs=[
        pl.BlockSpec((_MIX_TB, H), lambda i: (i, 0)),
        pl.BlockSpec((_MIX_TB, H), lambda i: (i + T // _MIX_TB, 0)),
        pl.BlockSpec((_MIX_TB, 2), lambda i: (i, 0)),
    ],
    out_specs=pl.BlockSpec((_MIX_TB, H), lambda i: (i, 0)),
    out_shape=jax.ShapeDtypeStruct((T, H), jnp.float32),
)


def kernel(hidden_states, W_router, w1, w2):
    Bv, Sv, Hv = hidden_states.shape
    x = hidden_states.reshape(Bv * Sv, Hv)
    inv, w01, te, tv = _plan(x, W_router)
    slots = jnp.concatenate([inv[:, 0], inv[:, 1]])
    toks = jnp.concatenate([jnp.arange(T, dtype=jnp.int32)] * 2)
    xs = _dispatch_kernel()(x, slots, toks)
    ys = _ffn(te.reshape(NT), tv.reshape(NT), xs, w1, w2)
    g = _combine_kernel()(ys, slots)
    out = _mix(g, g, w01)
    return out.reshape(Bv, Sv, Hv)


# speedup vs baseline: 1.9335x; 1.3087x over previous
import functools

import jax
import jax.numpy as jnp
from jax import lax
from jax.experimental import pallas as pl
from jax.experimental.pallas import tpu as pltpu
from jax.experimental.pallas import tpu_sc as plsc

E = 8
H = 1024
FFN = 2048
F2 = 2 * FFN
T = 2048
K = 2
NPAIR = K * T

TM = 128
NT = 40
P = NT * TM

NC = 2
NS = 16
NW = NC * NS
HALF = P // NC
SLOTS_W = HALF // NS
GCH = 80
CPW = NPAIR // NW
CCH = 64


def _plan_body(x_ref, wr_ref, inv_ref, w01_ref, te_ref, tv_ref):
    x = x_ref[...]
    wr = wr_ref[...]
    mix = jnp.dot(x, wr, preferred_element_type=jnp.float32)
    q, k, v = mix[:, 0:E], mix[:, E:2 * E], mix[:, 2 * E:3 * E]
    cols = []
    for i in range(E):
        a = q[:, i:i + 1] * k
        m = jnp.max(a, axis=1, keepdims=True)
        ex = jnp.exp(a - m)
        cols.append(jnp.sum(ex * v, axis=1, keepdims=True)
                    / jnp.sum(ex, axis=1, keepdims=True))
    logits = jnp.concatenate(cols, axis=1)
    iota8 = lax.broadcasted_iota(jnp.int32, (T, E), 1)
    l0 = jnp.max(logits, axis=1, keepdims=True)
    i0 = jnp.min(jnp.where(logits == l0, iota8, E), axis=1, keepdims=True)
    rest = jnp.where(iota8 == i0, -jnp.inf, logits)
    l1 = jnp.max(rest, axis=1, keepdims=True)
    i1 = jnp.min(jnp.where(rest == l1, iota8, E), axis=1, keepdims=True)
    w0 = 1.0 / (1.0 + jnp.exp(l1 - l0))

    oh0 = (iota8 == i0).astype(jnp.float32)
    oh1 = (iota8 == i1).astype(jnp.float32)
    assign = oh0 + oh1
    r = lax.broadcasted_iota(jnp.int32, (TM, TM), 0)
    c = lax.broadcasted_iota(jnp.int32, (TM, TM), 1)
    tri = (r >= c).astype(jnp.float32)
    carry = jnp.zeros((1, E), jnp.float32)
    parts = []
    for b in range(T // TM):
        cum = jnp.dot(tri, assign[b * TM:(b + 1) * TM, :],
                      preferred_element_type=jnp.float32) + carry
        parts.append(cum)
        carry = cum[TM - 1:TM, :]
    incl = jnp.concatenate(parts, axis=0)
    cnt = carry
    tiles = jnp.ceil(cnt / TM)
    ue = (lax.broadcasted_iota(jnp.int32, (E, E), 0)
          <= lax.broadcasted_iota(jnp.int32, (E, E), 1)).astype(jnp.float32)
    cumt = jnp.dot(tiles, ue, preferred_element_type=jnp.float32)
    start_slot = (cumt - tiles) * TM
    pos = start_slot + incl - 1.0
    inv0 = jnp.sum(oh0 * pos, axis=1, keepdims=True)
    inv1 = jnp.sum(oh1 * pos, axis=1, keepdims=True)
    inv_ref[...] = jnp.concatenate([inv0, inv1], axis=1).astype(jnp.int32)
    w01_ref[...] = jnp.concatenate([w0, 1.0 - w0], axis=1)
    jt = lax.broadcasted_iota(jnp.int32, (NT, E), 0).astype(jnp.float32)
    raw = jnp.sum((cumt <= jt).astype(jnp.float32), axis=1, keepdims=True)
    last_active = jnp.sum((cumt < cumt[0:1, E - 1:E]).astype(jnp.float32),
                          axis=1, keepdims=True)
    te_ref[...] = jnp.minimum(raw, last_active).astype(jnp.int32)
    tv_ref[...] = (jt[:, 0:1] < cumt[0:1, E - 1:E]).astype(jnp.int32)


_plan = pl.pallas_call(
    _plan_body,
    out_shape=[
        jax.ShapeDtypeStruct((T, 2), jnp.int32),
        jax.ShapeDtypeStruct((T, 2), jnp.float32),
        jax.ShapeDtypeStruct((NT, 1), jnp.int32),
        jax.ShapeDtypeStruct((NT, 1), jnp.int32),
    ],
)


@functools.cache
def _sc_mesh():
    return plsc.VectorSubcoreMesh(
        core_axis_name="c", subcore_axis_name="s",
        num_cores=NC, num_subcores=NS)


TPW = T // NW


@functools.cache
def _dispatch_kernel():
    @functools.partial(
        pl.kernel,
        out_type=jax.ShapeDtypeStruct((P, H), jnp.float32),
        mesh=_sc_mesh(),
        scratch_types=[
            pltpu.VMEM((K, TPW), jnp.int32),
            pltpu.VMEM((TPW, H), jnp.float32),
            pltpu.SemaphoreType.DMA,
        ],
        compiler_params=pltpu.CompilerParams(needs_layout_passes=False),
    )
    def dispatch(x_hbm, idx3_hbm, xs_hbm, idxw_v, rows_v, sem):
        c = lax.axis_index("c")
        s = lax.axis_index("s")
        wid = c * NS + s
        pltpu.sync_copy(x_hbm.at[pl.ds(wid * TPW, TPW)], rows_v)
        pltpu.sync_copy(idx3_hbm.at[wid], idxw_v)
        cps = [pltpu.async_copy(rows_v, xs_hbm.at[idxw_v.at[k]], sem)
               for k in range(K)]
        for cp in cps:
            cp.wait()

    return dispatch


def _ffn_body(te_ref, tv_ref, xs_ref, w1_ref, w2_ref, ys_ref):
    j = pl.program_id(0)

    @pl.when(tv_ref[j] == 1)
    def _():
        xb = xs_ref[...]
        h = jnp.dot(xb, w1_ref[0], preferred_element_type=jnp.float32)
        a = h[:, :FFN]
        b = h[:, FFN:]
        act = (a * lax.logistic(a)) * b
        ys_ref[...] = jnp.dot(act, w2_ref[0],
                              preferred_element_type=jnp.float32)


_ffn = pl.pallas_call(
    _ffn_body,
    grid_spec=pltpu.PrefetchScalarGridSpec(
        num_scalar_prefetch=2,
        grid=(NT,),
        in_specs=[
            pl.BlockSpec((TM, H), lambda j, te, tv: (j, 0)),
            pl.BlockSpec((1, H, F2), lambda j, te, tv: (te[j], 0, 0)),
            pl.BlockSpec((1, FFN, H), lambda j, te, tv: (te[j], 0, 0)),
        ],
        out_specs=pl.BlockSpec((TM, H), lambda j, te, tv: (j, 0)),
    ),
    out_shape=jax.ShapeDtypeStruct((P, H), jnp.float32),
    compiler_params=pltpu.CompilerParams(
        dimension_semantics=("arbitrary",)),
)


@functools.cache
def _combine_kernel():
    @functools.partial(
        pl.kernel,
        out_type=jax.ShapeDtypeStruct((NPAIR, H), jnp.float32),
        mesh=_sc_mesh(),
        scratch_types=[
            pltpu.VMEM((CPW,), jnp.int32),
            pltpu.VMEM((CCH, H), jnp.float32),
            pltpu.SemaphoreType.DMA,
        ],
    )
    def combine(ys_hbm, slots_hbm, g_hbm, idx_v, rows_v, sem):
        c = lax.axis_index("c")
        s = lax.axis_index("s")
        base = (s * NC + c) * CPW
        pltpu.sync_copy(slots_hbm.at[pl.ds(base, CPW)], idx_v)
        for ch in range(CPW // CCH):
            pltpu.async_copy(
                ys_hbm.at[idx_v.at[pl.ds(ch * CCH, CCH)]],
                rows_v, sem).wait()
            pltpu.sync_copy(rows_v, g_hbm.at[pl.ds(base + ch * CCH, CCH)])

    return combine


def _mix_body(g_ref, gg_ref, w_ref, o_ref):
    w = w_ref[...]
    o_ref[...] = g_ref[...] * w[:, 0:1] + gg_ref[...] * w[:, 1:2]


_MIX_TB = 256
_mix = pl.pallas_call(
    _mix_body,
    grid=(T // _MIX_TB,),
    in_specs=[
        pl.BlockSpec((_MIX_TB, H), lambda i: (i, 0)),
        pl.BlockSpec((_MIX_TB, H), lambda i: (i + T // _MIX_TB, 0)),
        pl.BlockSpec((_MIX_TB, 2), lambda i: (i, 0)),
    ],
    out_specs=pl.BlockSpec((_MIX_TB, H), lambda i: (i, 0)),
    out_shape=jax.ShapeDtypeStruct((T, H), jnp.float32),
)


def kernel(hidden_states, W_router, w1, w2):
    Bv, Sv, Hv = hidden_states.shape
    x = hidden_states.reshape(Bv * Sv, Hv)
    inv, w01, te, tv = _plan(x, W_router)
    slots = jnp.concatenate([inv[:, 0], inv[:, 1]])
    idx3 = inv.reshape(NW, TPW, K).transpose(0, 2, 1)
    xs = _dispatch_kernel()(x, idx3)
    ys = _ffn(te.reshape(NT), tv.reshape(NT), xs, w1, w2)
    g = _combine_kernel()(ys, slots)
    out = _mix(g, g, w01)
    return out.reshape(Bv, Sv, Hv)
